# baseline (speedup 1.0000x reference)
import functools

import jax
import jax.numpy as jnp
from jax import lax
from jax.experimental import pallas as pl
from jax.experimental.pallas import tpu as pltpu
from jax.experimental.pallas import tpu_sc as plsc

VOCAB = 1000000
DIM = 64
BATCH = 16384
HIST = 50

NC = 2
NS = 16
NW = NC * NS
ROWS_PER_W = BATCH // NW
NBUF = 6
UNROLL = 5

_mesh = plsc.VectorSubcoreMesh(core_axis_name="c", subcore_axis_name="s")


@functools.partial(
    pl.kernel,
    mesh=_mesh,
    compiler_params=pltpu.CompilerParams(use_tc_tiling_on_sc=False),
    out_type=jax.ShapeDtypeStruct((BATCH, DIM), jnp.float32),
    scratch_types=[
        pltpu.VMEM((ROWS_PER_W, HIST), jnp.int32),
        pltpu.VMEM((NBUF, HIST, DIM), jnp.float32),
        pltpu.VMEM((ROWS_PER_W, DIM), jnp.float32),
        pltpu.SemaphoreType.DMA((NBUF,)),
    ],
)
def _cbow_sc(idx_hbm, table_hbm, out_hbm, idx_v, bufs_v, out_v, sems):
    wid = lax.axis_index("s") * NC + lax.axis_index("c")
    row0 = wid * ROWS_PER_W

    pltpu.sync_copy(idx_hbm.at[pl.ds(row0, ROWS_PER_W)], idx_v)

    zero = jnp.zeros((16,), jnp.float32)

    for b in range(NBUF):
        pltpu.async_copy(table_hbm.at[idx_v.at[b]], bufs_v.at[b], sems.at[b])

    def group_body(g, _):
        for b in range(NBUF):
            r = g * NBUF + b
            buf = bufs_v.at[b]
            pltpu.make_async_copy(
                table_hbm.at[idx_v.at[r]], buf, sems.at[b]).wait()

            def h_body(h, accs, buf=buf):
                a0, a1, a2, a3 = accs
                for u in range(UNROLL):
                    hp = h * UNROLL + u
                    a0 = a0 + buf[hp, pl.ds(0, 16)]
                    a1 = a1 + buf[hp, pl.ds(16, 16)]
                    a2 = a2 + buf[hp, pl.ds(32, 16)]
                    a3 = a3 + buf[hp, pl.ds(48, 16)]
                return (a0, a1, a2, a3)

            a0, a1, a2, a3 = lax.fori_loop(
                0, HIST // UNROLL, h_body, (zero, zero, zero, zero))
            out_v[r, pl.ds(0, 16)] = a0
            out_v[r, pl.ds(16, 16)] = a1
            out_v[r, pl.ds(32, 16)] = a2
            out_v[r, pl.ds(48, 16)] = a3

            nxt = r + NBUF
            @pl.when(nxt < ROWS_PER_W)
            def _():
                pltpu.async_copy(
                    table_hbm.at[idx_v.at[nxt]], bufs_v.at[b], sems.at[b])
        return 0

    lax.fori_loop(0, ROWS_PER_W // NBUF, group_body, 0)

    for r in range(ROWS_PER_W - ROWS_PER_W % NBUF, ROWS_PER_W):
        b = r % NBUF
        buf = bufs_v.at[b]
        pltpu.make_async_copy(
            table_hbm.at[idx_v.at[r]], buf, sems.at[b]).wait()

        def h_body(h, accs, buf=buf):
            a0, a1, a2, a3 = accs
            for u in range(UNROLL):
                hp = h * UNROLL + u
                a0 = a0 + buf[hp, pl.ds(0, 16)]
                a1 = a1 + buf[hp, pl.ds(16, 16)]
                a2 = a2 + buf[hp, pl.ds(32, 16)]
                a3 = a3 + buf[hp, pl.ds(48, 16)]
            return (a0, a1, a2, a3)

        a0, a1, a2, a3 = lax.fori_loop(
            0, HIST // UNROLL, h_body, (zero, zero, zero, zero))
        out_v[r, pl.ds(0, 16)] = a0
        out_v[r, pl.ds(16, 16)] = a1
        out_v[r, pl.ds(32, 16)] = a2
        out_v[r, pl.ds(48, 16)] = a3

    pltpu.sync_copy(out_v, out_hbm.at[pl.ds(row0, ROWS_PER_W)])


def kernel(input_text, table):
    return _cbow_sc(input_text, table)
